# baseline (speedup 1.0000x reference)
import dataclasses
import functools

import jax
import jax.numpy as jnp
from jax import lax
from jax.experimental import pallas as pl
from jax.experimental.pallas import tpu as pltpu
from jax.experimental.pallas import tpu_sc as plsc

N = 10000
E = 320000
D = 128
DC = 64

NC = 2
NS = 16
NW = NC * NS
C = 96
NCH = 106
EP = NW * NCH * C
RPT = N // NS

_sc_mesh = plsc.VectorSubcoreMesh(
    core_axis_name="c", subcore_axis_name="s", num_cores=NC, num_subcores=NS)

_sc_params = pltpu.CompilerParams()
if "needs_layout_passes" in pltpu.CompilerParams.__dataclass_fields__:
    _sc_params = dataclasses.replace(_sc_params, needs_layout_passes=False)
if "use_tc_tiling_on_sc" in pltpu.CompilerParams.__dataclass_fields__:
    _sc_params = dataclasses.replace(_sc_params, use_tc_tiling_on_sc=False)


def _spmm_sc(table, e3):
    @functools.partial(
        pl.kernel,
        out_type=jax.ShapeDtypeStruct((NC, N, D), jnp.float32),
        mesh=_sc_mesh,
        scratch_types=[
            pltpu.VMEM_SHARED((N, D), jnp.float32),
            pltpu.VMEM((3, C), jnp.int32),
            pltpu.VMEM((3, C), jnp.int32),
            pltpu.VMEM((C,), jnp.int32),
            pltpu.VMEM((C,), jnp.int32),
            pltpu.VMEM((C, D // 2), jnp.int32),
            pltpu.VMEM((C, D // 2), jnp.int32),
            pltpu.VMEM((C, D), jnp.float32),
            pltpu.SemaphoreType.DMA,
            pltpu.SemaphoreType.DMA,
            pltpu.SemaphoreType.DMA,
            pltpu.SemaphoreType.DMA,
            pltpu.SemaphoreType.DMA,
        ],
        compiler_params=_sc_params,
    )
    def spmm_kernel(table_hbm, e3_hbm, out_hbm,
                    acc_sh, e3_a, e3_b, dc_a, dc_b, rows_a, rows_b, rf32,
                    si_a, si_b, sg_a, sg_b, ss):
        c = lax.axis_index("c")
        s = lax.axis_index("s")
        wid = c * NS + s

        @pl.loop(0, C)
        def _(r):
            for dd in range(D // 16):
                rf32[r, pl.ds(dd * 16, 16)] = jnp.zeros((16,), jnp.float32)

        for k in range(RPT // C):
            pltpu.sync_copy(rf32, acc_sh.at[pl.ds(s * RPT + k * C, C)])
        rem = RPT % C
        if rem:
            pltpu.sync_copy(rf32.at[pl.ds(0, rem)],
                            acc_sh.at[pl.ds(s * RPT + (RPT // C) * C, rem)])
        plsc.subcore_barrier()

        def process(j, e3_v, dc_v, rows_v, sg, si):
            pltpu.make_async_copy(table_hbm.at[e3_v.at[0]], rows_v, sg).wait()
            for g in range(C // 16):
                sl = pl.ds(g * 16, 16)
                dc_v[sl] = e3_v[1, sl]

            @pl.when(j > 0)
            def _():
                pltpu.make_async_copy(rf32, acc_sh.at[dc_v], ss).wait()

            @pl.loop(0, C // 16)
            def _(g):
                wv = lax.bitcast_convert_type(
                    e3_v[2, pl.ds(g * 16, 16)], jnp.float32)
                for k in range(16):
                    spl = jnp.full((16,), wv[k], jnp.float32)
                    e = g * 16 + k
                    for g2 in range(D // 32):
                        w32 = rows_v[e, pl.ds(g2 * 16, 16)]
                        lo = lax.bitcast_convert_type(
                            lax.shift_left(w32, 16), jnp.float32)
                        hi = lax.bitcast_convert_type(
                            lax.bitwise_and(w32, jnp.int32(-65536)),
                            jnp.float32)
                        rf32[e, pl.ds(g2 * 32, 16)] = lo * spl
                        rf32[e, pl.ds(g2 * 32 + 16, 16)] = hi * spl

            pltpu.async_copy(rf32, acc_sh.at[dc_v], ss, add=True)

            @pl.when(j + 2 < NCH)
            def _():
                pltpu.async_copy(e3_hbm.at[wid, j + 2], e3_v, si)

        def refill_gather(j, e3_v, rows_v, sg, si):
            @pl.when(j < NCH)
            def _():
                pltpu.make_async_copy(e3_hbm.at[wid, 0], e3_v, si).wait()
                pltpu.async_copy(table_hbm.at[e3_v.at[0]], rows_v, sg)

        pltpu.async_copy(e3_hbm.at[wid, 0], e3_a, si_a)
        pltpu.async_copy(e3_hbm.at[wid, 1], e3_b, si_b)
        pltpu.make_async_copy(e3_hbm.at[wid, 0], e3_a, si_a).wait()
        pltpu.async_copy(table_hbm.at[e3_a.at[0]], rows_a, sg_a)
        pltpu.make_async_copy(e3_hbm.at[wid, 1], e3_b, si_b).wait()
        pltpu.async_copy(table_hbm.at[e3_b.at[0]], rows_b, sg_b)

        @pl.loop(0, NCH // 2)
        def _(it):
            j0 = it * 2
            j1 = j0 + 1
            process(j0, e3_a, dc_a, rows_a, sg_a, si_a)
            process(j1, e3_b, dc_b, rows_b, sg_b, si_b)
            refill_gather(j0 + 2, e3_a, rows_a, sg_a, si_a)
            refill_gather(j1 + 2, e3_b, rows_b, sg_b, si_b)

        pltpu.make_async_copy(rf32, acc_sh.at[dc_b], ss).wait()
        plsc.subcore_barrier()

        WO = 624
        pltpu.sync_copy(acc_sh.at[pl.ds(s * WO, WO)],
                        out_hbm.at[c, pl.ds(s * WO, WO)])

        @pl.when(s == NS - 1)
        def _():
            pltpu.sync_copy(acc_sh.at[pl.ds(NS * WO, N - NS * WO)],
                            out_hbm.at[c, pl.ds(NS * WO, N - NS * WO)])

    return spmm_kernel(table, e3)


_BM = 1000


def _interleave_bf16(r):
    r = r.reshape(r.shape[0], D // 32, 2, 16).swapaxes(2, 3)
    return r.reshape(r.shape[0], D).astype(jnp.bfloat16)


def _pack_i32(t):
    return lax.bitcast_convert_type(t.reshape(N, D // 2, 2), jnp.int32)


def _mm1_body(x_ref, w_ref, o_ref):
    o_ref[...] = _interleave_bf16(
        jnp.dot(x_ref[...], w_ref[...], preferred_element_type=jnp.float32))


def _mm1(x, W1):
    return pl.pallas_call(
        _mm1_body,
        grid=(N // _BM,),
        in_specs=[
            pl.BlockSpec((_BM, D), lambda i: (i, 0)),
            pl.BlockSpec((D, D), lambda i: (0, 0)),
        ],
        out_specs=pl.BlockSpec((_BM, D), lambda i: (i, 0)),
        out_shape=jax.ShapeDtypeStruct((N, D), jnp.bfloat16),
    )(x, W1)


def _combine_relu_body(p_ref, b_ref, o_ref):
    o_ref[...] = _interleave_bf16(
        jnp.maximum(p_ref[0] + p_ref[1] + b_ref[...], 0.0))


def _combine_relu(p, b1):
    return pl.pallas_call(
        _combine_relu_body,
        grid=(N // _BM,),
        in_specs=[
            pl.BlockSpec((NC, _BM, D), lambda i: (0, i, 0)),
            pl.BlockSpec((1, D), lambda i: (0, 0)),
        ],
        out_specs=pl.BlockSpec((_BM, D), lambda i: (i, 0)),
        out_shape=jax.ShapeDtypeStruct((N, D), jnp.bfloat16),
    )(p, b1.reshape(1, D))


def _final_body(q_ref, w_ref, b_ref, o_ref):
    t = q_ref[0] + q_ref[1]
    o = jnp.dot(t, w_ref[...], preferred_element_type=jnp.float32) + b_ref[...]
    m = jnp.max(o, axis=1, keepdims=True)
    ex = jnp.exp(o - m)
    lse = jnp.log(jnp.sum(ex, axis=1, keepdims=True)) + m
    o_ref[...] = o - lse


def _final(q, W2, b2):
    return pl.pallas_call(
        _final_body,
        grid=(N // _BM,),
        in_specs=[
            pl.BlockSpec((NC, _BM, D), lambda i: (0, i, 0)),
            pl.BlockSpec((D, DC), lambda i: (0, 0)),
            pl.BlockSpec((1, DC), lambda i: (0, 0)),
        ],
        out_specs=pl.BlockSpec((_BM, DC), lambda i: (i, 0)),
        out_shape=jax.ShapeDtypeStruct((N, DC), jnp.float32),
    )(q, W2, b2.reshape(1, DC))


def kernel(x, edge_index, edge_weight, W1, b1, W2, b2):
    pad = EP - E
    pad_idx = (jnp.arange(pad, dtype=jnp.int32) * 8) % N
    src_p = jnp.concatenate([edge_index[0], pad_idx]).reshape(NW, NCH, 1, C)
    dst_p = jnp.concatenate([edge_index[1], pad_idx]).reshape(NW, NCH, 1, C)
    w_bits = lax.bitcast_convert_type(
        jnp.concatenate([edge_weight, jnp.zeros((pad,), jnp.float32)]),
        jnp.int32).reshape(NW, NCH, 1, C)
    e3 = jnp.concatenate([src_p, dst_p, w_bits], axis=2)

    support = _pack_i32(_mm1(x, W1))
    p = _spmm_sc(support, e3)
    h = _pack_i32(_combine_relu(p, b1))
    q = _spmm_sc(h, e3)
    return _final(q, W2, b2)


# speedup vs baseline: 3.0616x; 3.0616x over previous
import functools

import jax
import jax.numpy as jnp
from jax import lax
from jax.experimental import pallas as pl
from jax.experimental.pallas import tpu as pltpu
from jax.experimental.pallas import tpu_sc as plsc

N = 10000
E = 320000
D = 128
DC = 64

NC = 2
NS = 16
NW = NC * NS
C = 64
NB = 4
NCH = 160
EP = NW * NCH * C
RPT = N // NS

_sc_mesh = plsc.VectorSubcoreMesh(
    core_axis_name="c", subcore_axis_name="s", num_cores=NC, num_subcores=NS)


def _spmm_sc(table, e3):
    @functools.partial(
        pl.kernel,
        out_type=jax.ShapeDtypeStruct((NC, N, D), jnp.float32),
        mesh=_sc_mesh,
        scratch_types=[
            pltpu.VMEM_SHARED((N, D), jnp.float32),
        ] + [pltpu.VMEM((3, C), jnp.int32)] * NB
          + [pltpu.VMEM((C,), jnp.int32)] * NB
          + [pltpu.VMEM((C, D), jnp.float32)] * NB
          + [pltpu.SemaphoreType.DMA] * (3 * NB),
    )
    def spmm_kernel(table_hbm, e3_hbm, out_hbm, acc_sh, *bufs):
        e3_v = bufs[0:NB]
        dc_v = bufs[NB:2 * NB]
        rows_v = bufs[2 * NB:3 * NB]
        si = bufs[3 * NB:4 * NB]
        sg = bufs[4 * NB:5 * NB]
        ss = bufs[5 * NB:6 * NB]

        c = lax.axis_index("c")
        s = lax.axis_index("s")
        wid = c * NS + s

        @pl.loop(0, C)
        def _(r):
            for dd in range(D // 16):
                rows_v[0][r, pl.ds(dd * 16, 16)] = jnp.zeros((16,), jnp.float32)

        for k in range(RPT // C):
            pltpu.sync_copy(rows_v[0], acc_sh.at[pl.ds(s * RPT + k * C, C)])
        rem = RPT % C
        if rem:
            pltpu.sync_copy(rows_v[0].at[pl.ds(0, rem)],
                            acc_sh.at[pl.ds(s * RPT + (RPT // C) * C, rem)])
        plsc.subcore_barrier()

        def process(j, b):
            pltpu.make_async_copy(
                table_hbm.at[e3_v[b].at[0]], rows_v[b], sg[b]).wait()
            for g in range(C // 16):
                sl = pl.ds(g * 16, 16)
                dc_v[b][sl] = e3_v[b][1, sl]

            @pl.loop(0, C // 16)
            def _(g):
                wv = lax.bitcast_convert_type(
                    e3_v[b][2, pl.ds(g * 16, 16)], jnp.float32)
                for k in range(16):
                    spl = jnp.full((16,), wv[k], jnp.float32)
                    e = g * 16 + k
                    for dd in range(D // 16):
                        sl2 = pl.ds(dd * 16, 16)
                        rows_v[b][e, sl2] = rows_v[b][e, sl2] * spl

            pltpu.async_copy(rows_v[b], acc_sh.at[dc_v[b]], ss[b], add=True)

            @pl.when(j + NB < NCH)
            def _():
                pltpu.async_copy(e3_hbm.at[wid, j + NB], e3_v[b], si[b])

        def refill_gather(j, b):
            @pl.when(j < NCH)
            def _():
                pltpu.make_async_copy(rows_v[b], acc_sh.at[dc_v[b]],
                                      ss[b]).wait()
                pltpu.make_async_copy(e3_hbm.at[wid, 0], e3_v[b], si[b]).wait()
                pltpu.async_copy(table_hbm.at[e3_v[b].at[0]], rows_v[b], sg[b])

        for b in range(NB):
            pltpu.async_copy(e3_hbm.at[wid, b], e3_v[b], si[b])
        for b in range(NB):
            pltpu.make_async_copy(e3_hbm.at[wid, 0], e3_v[b], si[b]).wait()
            pltpu.async_copy(table_hbm.at[e3_v[b].at[0]], rows_v[b], sg[b])

        @pl.loop(0, NCH // NB)
        def _(it):
            j0 = it * NB
            for b in range(NB):
                process(j0 + b, b)
            for b in range(NB):
                refill_gather(j0 + b + NB, b)

        for b in range(NB):
            pltpu.make_async_copy(rows_v[b], acc_sh.at[dc_v[b]], ss[b]).wait()
        plsc.subcore_barrier()

        WO = 624
        pltpu.sync_copy(acc_sh.at[pl.ds(s * WO, WO)],
                        out_hbm.at[c, pl.ds(s * WO, WO)])

        @pl.when(s == NS - 1)
        def _():
            pltpu.sync_copy(acc_sh.at[pl.ds(NS * WO, N - NS * WO)],
                            out_hbm.at[c, pl.ds(NS * WO, N - NS * WO)])

    return spmm_kernel(table, e3)


_BM = 1000


def _mm1_body(x_ref, w_ref, o_ref):
    o_ref[...] = jnp.dot(x_ref[...], w_ref[...],
                         preferred_element_type=jnp.float32)


def _mm1(x, W1):
    return pl.pallas_call(
        _mm1_body,
        grid=(N // _BM,),
        in_specs=[
            pl.BlockSpec((_BM, D), lambda i: (i, 0)),
            pl.BlockSpec((D, D), lambda i: (0, 0)),
        ],
        out_specs=pl.BlockSpec((_BM, D), lambda i: (i, 0)),
        out_shape=jax.ShapeDtypeStruct((N, D), jnp.float32),
    )(x, W1)


def _combine_relu_body(p_ref, b_ref, o_ref):
    o_ref[...] = jnp.maximum(p_ref[0] + p_ref[1] + b_ref[...], 0.0)


def _combine_relu(p, b1):
    return pl.pallas_call(
        _combine_relu_body,
        grid=(N // _BM,),
        in_specs=[
            pl.BlockSpec((NC, _BM, D), lambda i: (0, i, 0)),
            pl.BlockSpec((1, D), lambda i: (0, 0)),
        ],
        out_specs=pl.BlockSpec((_BM, D), lambda i: (i, 0)),
        out_shape=jax.ShapeDtypeStruct((N, D), jnp.float32),
    )(p, b1.reshape(1, D))


def _final_body(q_ref, w_ref, b_ref, o_ref):
    t = q_ref[0] + q_ref[1]
    o = jnp.dot(t, w_ref[...], preferred_element_type=jnp.float32) + b_ref[...]
    m = jnp.max(o, axis=1, keepdims=True)
    ex = jnp.exp(o - m)
    lse = jnp.log(jnp.sum(ex, axis=1, keepdims=True)) + m
    o_ref[...] = o - lse


def _final(q, W2, b2):
    return pl.pallas_call(
        _final_body,
        grid=(N // _BM,),
        in_specs=[
            pl.BlockSpec((NC, _BM, D), lambda i: (0, i, 0)),
            pl.BlockSpec((D, DC), lambda i: (0, 0)),
            pl.BlockSpec((1, DC), lambda i: (0, 0)),
        ],
        out_specs=pl.BlockSpec((_BM, DC), lambda i: (i, 0)),
        out_shape=jax.ShapeDtypeStruct((N, DC), jnp.float32),
    )(q, W2, b2.reshape(1, DC))


def kernel(x, edge_index, edge_weight, W1, b1, W2, b2):
    pad = EP - E
    pad_idx = (jnp.arange(pad, dtype=jnp.int32) * 8) % N
    src_p = jnp.concatenate([edge_index[0], pad_idx]).reshape(NW, NCH, 1, C)
    dst_p = jnp.concatenate([edge_index[1], pad_idx]).reshape(NW, NCH, 1, C)
    w_bits = lax.bitcast_convert_type(
        jnp.concatenate([edge_weight, jnp.zeros((pad,), jnp.float32)]),
        jnp.int32).reshape(NW, NCH, 1, C)
    e3 = jnp.concatenate([src_p, dst_p, w_bits], axis=2)

    support = _mm1(x, W1)
    p = _spmm_sc(support, e3)
    h = _combine_relu(p, b1)
    q = _spmm_sc(h, e3)
    return _final(q, W2, b2)


# speedup vs baseline: 3.1540x; 1.0302x over previous
import functools

import jax
import jax.numpy as jnp
from jax import lax
from jax.experimental import pallas as pl
from jax.experimental.pallas import tpu as pltpu
from jax.experimental.pallas import tpu_sc as plsc

N = 10000
E = 320000
D = 128
DC = 64

NC = 2
NS = 16
NW = NC * NS
C = 80
NB = 4
NCH = 128
EP = NW * NCH * C
RPT = N // NS

_sc_mesh = plsc.VectorSubcoreMesh(
    core_axis_name="c", subcore_axis_name="s", num_cores=NC, num_subcores=NS)


def _spmm_sc(table, e3):
    @functools.partial(
        pl.kernel,
        out_type=jax.ShapeDtypeStruct((NC, N, D), jnp.float32),
        mesh=_sc_mesh,
        scratch_types=[
            pltpu.VMEM_SHARED((N, D), jnp.float32),
        ] + [pltpu.VMEM((3, C), jnp.int32)] * NB
          + [pltpu.VMEM((C,), jnp.int32)] * NB
          + [pltpu.VMEM((C, D), jnp.float32)] * NB
          + [pltpu.SemaphoreType.DMA] * (3 * NB),
    )
    def spmm_kernel(table_hbm, e3_hbm, out_hbm, acc_sh, *bufs):
        e3_v = bufs[0:NB]
        dc_v = bufs[NB:2 * NB]
        rows_v = bufs[2 * NB:3 * NB]
        si = bufs[3 * NB:4 * NB]
        sg = bufs[4 * NB:5 * NB]
        ss = bufs[5 * NB:6 * NB]

        c = lax.axis_index("c")
        s = lax.axis_index("s")
        wid = c * NS + s

        @pl.loop(0, C)
        def _(r):
            for dd in range(D // 16):
                rows_v[0][r, pl.ds(dd * 16, 16)] = jnp.zeros((16,), jnp.float32)

        for k in range(RPT // C):
            pltpu.sync_copy(rows_v[0], acc_sh.at[pl.ds(s * RPT + k * C, C)])
        rem = RPT % C
        if rem:
            pltpu.sync_copy(rows_v[0].at[pl.ds(0, rem)],
                            acc_sh.at[pl.ds(s * RPT + (RPT // C) * C, rem)])
        plsc.subcore_barrier()

        def process(j, b):
            pltpu.make_async_copy(
                table_hbm.at[e3_v[b].at[0]], rows_v[b], sg[b]).wait()
            for g in range(C // 16):
                sl = pl.ds(g * 16, 16)
                dc_v[b][sl] = e3_v[b][1, sl]

            @pl.loop(0, C // 16)
            def _(g):
                wv = lax.bitcast_convert_type(
                    e3_v[b][2, pl.ds(g * 16, 16)], jnp.float32)
                for k in range(16):
                    spl = jnp.full((16,), wv[k], jnp.float32)
                    e = g * 16 + k
                    for dd in range(D // 16):
                        sl2 = pl.ds(dd * 16, 16)
                        rows_v[b][e, sl2] = rows_v[b][e, sl2] * spl

            pltpu.async_copy(rows_v[b], acc_sh.at[dc_v[b]], ss[b], add=True)

            @pl.when(j + NB < NCH)
            def _():
                pltpu.async_copy(e3_hbm.at[wid, j + NB], e3_v[b], si[b])

        def refill_gather(j, b):
            @pl.when(j < NCH)
            def _():
                pltpu.make_async_copy(rows_v[b], acc_sh.at[dc_v[b]],
                                      ss[b]).wait()
                pltpu.make_async_copy(e3_hbm.at[wid, 0], e3_v[b], si[b]).wait()
                pltpu.async_copy(table_hbm.at[e3_v[b].at[0]], rows_v[b], sg[b])

        for b in range(NB):
            pltpu.async_copy(e3_hbm.at[wid, b], e3_v[b], si[b])
        for b in range(NB):
            pltpu.make_async_copy(e3_hbm.at[wid, 0], e3_v[b], si[b]).wait()
            pltpu.async_copy(table_hbm.at[e3_v[b].at[0]], rows_v[b], sg[b])

        @pl.loop(0, NCH // NB)
        def _(it):
            j0 = it * NB
            for b in range(NB):
                process(j0 + b, b)
            for b in range(NB):
                refill_gather(j0 + b + NB, b)

        for b in range(NB):
            pltpu.make_async_copy(rows_v[b], acc_sh.at[dc_v[b]], ss[b]).wait()
        plsc.subcore_barrier()

        WO = 624
        pltpu.sync_copy(acc_sh.at[pl.ds(s * WO, WO)],
                        out_hbm.at[c, pl.ds(s * WO, WO)])

        @pl.when(s == NS - 1)
        def _():
            pltpu.sync_copy(acc_sh.at[pl.ds(NS * WO, N - NS * WO)],
                            out_hbm.at[c, pl.ds(NS * WO, N - NS * WO)])

    return spmm_kernel(table, e3)


_BM = 1000


def _mm1_body(x_ref, w_ref, o_ref):
    o_ref[...] = jnp.dot(x_ref[...], w_ref[...],
                         preferred_element_type=jnp.float32)


def _mm1(x, W1):
    return pl.pallas_call(
        _mm1_body,
        grid=(N // _BM,),
        in_specs=[
            pl.BlockSpec((_BM, D), lambda i: (i, 0)),
            pl.BlockSpec((D, D), lambda i: (0, 0)),
        ],
        out_specs=pl.BlockSpec((_BM, D), lambda i: (i, 0)),
        out_shape=jax.ShapeDtypeStruct((N, D), jnp.float32),
    )(x, W1)


def _combine_relu_body(p_ref, b_ref, o_ref):
    o_ref[...] = jnp.maximum(p_ref[0] + p_ref[1] + b_ref[...], 0.0)


def _combine_relu(p, b1):
    return pl.pallas_call(
        _combine_relu_body,
        grid=(N // _BM,),
        in_specs=[
            pl.BlockSpec((NC, _BM, D), lambda i: (0, i, 0)),
            pl.BlockSpec((1, D), lambda i: (0, 0)),
        ],
        out_specs=pl.BlockSpec((_BM, D), lambda i: (i, 0)),
        out_shape=jax.ShapeDtypeStruct((N, D), jnp.float32),
    )(p, b1.reshape(1, D))


def _final_body(q_ref, w_ref, b_ref, o_ref):
    t = q_ref[0] + q_ref[1]
    o = jnp.dot(t, w_ref[...], preferred_element_type=jnp.float32) + b_ref[...]
    m = jnp.max(o, axis=1, keepdims=True)
    ex = jnp.exp(o - m)
    lse = jnp.log(jnp.sum(ex, axis=1, keepdims=True)) + m
    o_ref[...] = o - lse


def _final(q, W2, b2):
    return pl.pallas_call(
        _final_body,
        grid=(N // _BM,),
        in_specs=[
            pl.BlockSpec((NC, _BM, D), lambda i: (0, i, 0)),
            pl.BlockSpec((D, DC), lambda i: (0, 0)),
            pl.BlockSpec((1, DC), lambda i: (0, 0)),
        ],
        out_specs=pl.BlockSpec((_BM, DC), lambda i: (i, 0)),
        out_shape=jax.ShapeDtypeStruct((N, DC), jnp.float32),
    )(q, W2, b2.reshape(1, DC))


def kernel(x, edge_index, edge_weight, W1, b1, W2, b2):
    pad = EP - E
    pad_idx = (jnp.arange(pad, dtype=jnp.int32) * 8) % N
    src_p = jnp.concatenate([edge_index[0], pad_idx]).reshape(NW, NCH, 1, C)
    dst_p = jnp.concatenate([edge_index[1], pad_idx]).reshape(NW, NCH, 1, C)
    w_bits = lax.bitcast_convert_type(
        jnp.concatenate([edge_weight, jnp.zeros((pad,), jnp.float32)]),
        jnp.int32).reshape(NW, NCH, 1, C)
    e3 = jnp.concatenate([src_p, dst_p, w_bits], axis=2)

    support = _mm1(x, W1)
    p = _spmm_sc(support, e3)
    h = _combine_relu(p, b1)
    q = _spmm_sc(h, e3)
    return _final(q, W2, b2)


# speedup vs baseline: 3.1591x; 1.0016x over previous
import functools

import jax
import jax.numpy as jnp
from jax import lax
from jax.experimental import pallas as pl
from jax.experimental.pallas import tpu as pltpu
from jax.experimental.pallas import tpu_sc as plsc

N = 10000
E = 320000
D = 128
DC = 64

NC = 2
NS = 16
NW = NC * NS
C = 80
NB = 4
NCH = 128
EP = NW * NCH * C
RPT = N // NS

_sc_mesh = plsc.VectorSubcoreMesh(
    core_axis_name="c", subcore_axis_name="s", num_cores=NC, num_subcores=NS)


def _spmm_sc(table, e3):
    @functools.partial(
        pl.kernel,
        out_type=jax.ShapeDtypeStruct((NC, N, D), jnp.float32),
        mesh=_sc_mesh,
        scratch_types=[
            pltpu.VMEM_SHARED((N, D), jnp.float32),
        ] + [pltpu.VMEM((3, C), jnp.int32)] * NB
          + [pltpu.VMEM((C,), jnp.int32)] * NB
          + [pltpu.VMEM((C, D), jnp.float32)] * NB
          + [pltpu.SemaphoreType.DMA] * (3 * NB),
    )
    def spmm_kernel(table_hbm, e3_hbm, out_hbm, acc_sh, *bufs):
        e3_v = bufs[0:NB]
        dc_v = bufs[NB:2 * NB]
        rows_v = bufs[2 * NB:3 * NB]
        si = bufs[3 * NB:4 * NB]
        sg = bufs[4 * NB:5 * NB]
        ss = bufs[5 * NB:6 * NB]

        c = lax.axis_index("c")
        s = lax.axis_index("s")
        wid = c * NS + s

        @pl.loop(0, C)
        def _(r):
            for dd in range(D // 16):
                rows_v[0][r, pl.ds(dd * 16, 16)] = jnp.zeros((16,), jnp.float32)

        rem = RPT % C
        for k in range(RPT // C):
            pltpu.async_copy(rows_v[0], acc_sh.at[pl.ds(s * RPT + k * C, C)],
                             sg[0])
        if rem:
            pltpu.async_copy(rows_v[0].at[pl.ds(0, rem)],
                             acc_sh.at[pl.ds(s * RPT + (RPT // C) * C, rem)],
                             sg[0])
        for k in range(RPT // C):
            pltpu.make_async_copy(rows_v[0],
                                  acc_sh.at[pl.ds(s * RPT + k * C, C)],
                                  sg[0]).wait()
        if rem:
            pltpu.make_async_copy(rows_v[0].at[pl.ds(0, rem)],
                                  acc_sh.at[pl.ds(s * RPT + (RPT // C) * C,
                                                  rem)],
                                  sg[0]).wait()
        plsc.subcore_barrier()

        def process(j, b):
            pltpu.make_async_copy(
                table_hbm.at[e3_v[b].at[0]], rows_v[b], sg[b]).wait()
            for g in range(C // 16):
                sl = pl.ds(g * 16, 16)
                dc_v[b][sl] = e3_v[b][1, sl]

            @pl.loop(0, C // 16)
            def _(g):
                wv = lax.bitcast_convert_type(
                    e3_v[b][2, pl.ds(g * 16, 16)], jnp.float32)
                for k in range(16):
                    spl = jnp.full((16,), wv[k], jnp.float32)
                    e = g * 16 + k
                    for dd in range(D // 16):
                        sl2 = pl.ds(dd * 16, 16)
                        rows_v[b][e, sl2] = rows_v[b][e, sl2] * spl

            pltpu.async_copy(rows_v[b], acc_sh.at[dc_v[b]], ss[b], add=True)

            @pl.when(j + NB < NCH)
            def _():
                pltpu.async_copy(e3_hbm.at[wid, j + NB], e3_v[b], si[b])

        def refill_gather(j, b):
            @pl.when(j < NCH)
            def _():
                pltpu.make_async_copy(rows_v[b], acc_sh.at[dc_v[b]],
                                      ss[b]).wait()
                pltpu.make_async_copy(e3_hbm.at[wid, 0], e3_v[b], si[b]).wait()
                pltpu.async_copy(table_hbm.at[e3_v[b].at[0]], rows_v[b], sg[b])

        for b in range(NB):
            pltpu.async_copy(e3_hbm.at[wid, b], e3_v[b], si[b])
        for b in range(NB):
            pltpu.make_async_copy(e3_hbm.at[wid, 0], e3_v[b], si[b]).wait()
            pltpu.async_copy(table_hbm.at[e3_v[b].at[0]], rows_v[b], sg[b])

        @pl.loop(0, NCH // NB)
        def _(it):
            j0 = it * NB
            for b in range(NB):
                process(j0 + b, b)
            for b in range(NB):
                refill_gather(j0 + b + NB, b)

        for b in range(NB):
            pltpu.make_async_copy(rows_v[b], acc_sh.at[dc_v[b]], ss[b]).wait()
        plsc.subcore_barrier()

        WO = 624
        pltpu.sync_copy(acc_sh.at[pl.ds(s * WO, WO)],
                        out_hbm.at[c, pl.ds(s * WO, WO)])

        @pl.when(s == NS - 1)
        def _():
            pltpu.sync_copy(acc_sh.at[pl.ds(NS * WO, N - NS * WO)],
                            out_hbm.at[c, pl.ds(NS * WO, N - NS * WO)])

    return spmm_kernel(table, e3)


_BM = 1000


def _mm1_body(x_ref, w_ref, o_ref):
    o_ref[...] = jnp.dot(x_ref[...], w_ref[...],
                         preferred_element_type=jnp.float32)


def _mm1(x, W1):
    return pl.pallas_call(
        _mm1_body,
        grid=(N // _BM,),
        in_specs=[
            pl.BlockSpec((_BM, D), lambda i: (i, 0)),
            pl.BlockSpec((D, D), lambda i: (0, 0)),
        ],
        out_specs=pl.BlockSpec((_BM, D), lambda i: (i, 0)),
        out_shape=jax.ShapeDtypeStruct((N, D), jnp.float32),
    )(x, W1)


def _combine_relu_body(p_ref, b_ref, o_ref):
    o_ref[...] = jnp.maximum(p_ref[0] + p_ref[1] + b_ref[...], 0.0)


def _combine_relu(p, b1):
    return pl.pallas_call(
        _combine_relu_body,
        grid=(N // _BM,),
        in_specs=[
            pl.BlockSpec((NC, _BM, D), lambda i: (0, i, 0)),
            pl.BlockSpec((1, D), lambda i: (0, 0)),
        ],
        out_specs=pl.BlockSpec((_BM, D), lambda i: (i, 0)),
        out_shape=jax.ShapeDtypeStruct((N, D), jnp.float32),
    )(p, b1.reshape(1, D))


def _final_body(q_ref, w_ref, b_ref, o_ref):
    t = q_ref[0] + q_ref[1]
    o = jnp.dot(t, w_ref[...], preferred_element_type=jnp.float32) + b_ref[...]
    m = jnp.max(o, axis=1, keepdims=True)
    ex = jnp.exp(o - m)
    lse = jnp.log(jnp.sum(ex, axis=1, keepdims=True)) + m
    o_ref[...] = o - lse


def _final(q, W2, b2):
    return pl.pallas_call(
        _final_body,
        grid=(N // _BM,),
        in_specs=[
            pl.BlockSpec((NC, _BM, D), lambda i: (0, i, 0)),
            pl.BlockSpec((D, DC), lambda i: (0, 0)),
            pl.BlockSpec((1, DC), lambda i: (0, 0)),
        ],
        out_specs=pl.BlockSpec((_BM, DC), lambda i: (i, 0)),
        out_shape=jax.ShapeDtypeStruct((N, DC), jnp.float32),
    )(q, W2, b2.reshape(1, DC))


def kernel(x, edge_index, edge_weight, W1, b1, W2, b2):
    pad = EP - E
    pad_idx = (jnp.arange(pad, dtype=jnp.int32) * 8) % N
    src_p = jnp.concatenate([edge_index[0], pad_idx]).reshape(NW, NCH, 1, C)
    dst_p = jnp.concatenate([edge_index[1], pad_idx]).reshape(NW, NCH, 1, C)
    w_bits = lax.bitcast_convert_type(
        jnp.concatenate([edge_weight, jnp.zeros((pad,), jnp.float32)]),
        jnp.int32).reshape(NW, NCH, 1, C)
    e3 = jnp.concatenate([src_p, dst_p, w_bits], axis=2)

    support = _mm1(x, W1)
    p = _spmm_sc(support, e3)
    h = _combine_relu(p, b1)
    q = _spmm_sc(h, e3)
    return _final(q, W2, b2)


# speedup vs baseline: 3.2360x; 1.0243x over previous
import functools

import jax
import jax.numpy as jnp
from jax import lax
from jax.experimental import pallas as pl
from jax.experimental.pallas import tpu as pltpu
from jax.experimental.pallas import tpu_sc as plsc

N = 10000
E = 320000
D = 128
DC = 64

NC = 2
NS = 16
NW = NC * NS
C = 80
NB = 4
NCH = 128
EP = NW * NCH * C
RPT = N // NS

_sc_mesh = plsc.VectorSubcoreMesh(
    core_axis_name="c", subcore_axis_name="s", num_cores=NC, num_subcores=NS)


def _spmm_sc(table, e3):
    @functools.partial(
        pl.kernel,
        out_type=jax.ShapeDtypeStruct((NC, N, D), jnp.float32),
        mesh=_sc_mesh,
        scratch_types=[
            pltpu.VMEM_SHARED((N, D), jnp.float32),
        ] + [pltpu.VMEM((3, C), jnp.int32)] * NB
          + [pltpu.VMEM((C,), jnp.int32)] * NB
          + [pltpu.VMEM((C, D), jnp.float32)] * NB
          + [pltpu.SemaphoreType.DMA] * (3 * NB),
    )
    def spmm_kernel(table_hbm, e3_hbm, out_hbm, acc_sh, *bufs):
        e3_v = bufs[0:NB]
        dc_v = bufs[NB:2 * NB]
        rows_v = bufs[2 * NB:3 * NB]
        si = bufs[3 * NB:4 * NB]
        sg = bufs[4 * NB:5 * NB]
        ss = bufs[5 * NB:6 * NB]

        c = lax.axis_index("c")
        s = lax.axis_index("s")
        wid = c * NS + s

        @pl.loop(0, C)
        def _(r):
            for dd in range(D // 16):
                rows_v[0][r, pl.ds(dd * 16, 16)] = jnp.zeros((16,), jnp.float32)

        rem = RPT % C
        for k in range(RPT // C):
            pltpu.async_copy(rows_v[0], acc_sh.at[pl.ds(s * RPT + k * C, C)],
                             sg[0])
        if rem:
            pltpu.async_copy(rows_v[0].at[pl.ds(0, rem)],
                             acc_sh.at[pl.ds(s * RPT + (RPT // C) * C, rem)],
                             sg[0])
        for k in range(RPT // C):
            pltpu.make_async_copy(rows_v[0],
                                  acc_sh.at[pl.ds(s * RPT + k * C, C)],
                                  sg[0]).wait()
        if rem:
            pltpu.make_async_copy(rows_v[0].at[pl.ds(0, rem)],
                                  acc_sh.at[pl.ds(s * RPT + (RPT // C) * C,
                                                  rem)],
                                  sg[0]).wait()
        plsc.subcore_barrier()

        def process(j, b):
            pltpu.make_async_copy(
                table_hbm.at[e3_v[b].at[0]], rows_v[b], sg[b]).wait()
            for g in range(C // 16):
                sl = pl.ds(g * 16, 16)
                dc_v[b][sl] = e3_v[b][1, sl]

            @pl.loop(0, C // 16)
            def _(g):
                wv = lax.bitcast_convert_type(
                    e3_v[b][2, pl.ds(g * 16, 16)], jnp.float32)
                for k in range(16):
                    spl = jnp.full((16,), wv[k], jnp.float32)
                    e = g * 16 + k
                    for dd in range(D // 16):
                        sl2 = pl.ds(dd * 16, 16)
                        rows_v[b][e, sl2] = rows_v[b][e, sl2] * spl

            pltpu.async_copy(rows_v[b], acc_sh.at[dc_v[b]], ss[b], add=True)

            @pl.when(j + NB < NCH)
            def _():
                pltpu.async_copy(e3_hbm.at[wid, j + NB], e3_v[b], si[b])

        def refill_gather(j, b):
            @pl.when(j < NCH)
            def _():
                pltpu.make_async_copy(rows_v[b], acc_sh.at[dc_v[b]],
                                      ss[b]).wait()
                pltpu.make_async_copy(e3_hbm.at[wid, 0], e3_v[b], si[b]).wait()
                pltpu.async_copy(table_hbm.at[e3_v[b].at[0]], rows_v[b], sg[b])

        for b in range(NB):
            pltpu.async_copy(e3_hbm.at[wid, b], e3_v[b], si[b])
        for b in range(NB):
            pltpu.make_async_copy(e3_hbm.at[wid, 0], e3_v[b], si[b]).wait()
            pltpu.async_copy(table_hbm.at[e3_v[b].at[0]], rows_v[b], sg[b])

        @pl.loop(0, NCH // NB)
        def _(it):
            j0 = it * NB
            for b in range(NB):
                process(j0 + b, b)
            for b in range(NB):
                refill_gather(j0 + b + NB, b)

        for b in range(NB):
            pltpu.make_async_copy(rows_v[b], acc_sh.at[dc_v[b]], ss[b]).wait()
        plsc.subcore_barrier()

        WO = 624
        pltpu.sync_copy(acc_sh.at[pl.ds(s * WO, WO)],
                        out_hbm.at[c, pl.ds(s * WO, WO)])

        @pl.when(s == NS - 1)
        def _():
            pltpu.sync_copy(acc_sh.at[pl.ds(NS * WO, N - NS * WO)],
                            out_hbm.at[c, pl.ds(NS * WO, N - NS * WO)])

    return spmm_kernel(table, e3)


_BM = 2000


def _mm1_body(x_ref, w_ref, o_ref):
    o_ref[...] = jnp.dot(x_ref[...], w_ref[...],
                         preferred_element_type=jnp.float32)


def _mm1(x, W1):
    return pl.pallas_call(
        _mm1_body,
        grid=(N // _BM,),
        in_specs=[
            pl.BlockSpec((_BM, D), lambda i: (i, 0)),
            pl.BlockSpec((D, D), lambda i: (0, 0)),
        ],
        out_specs=pl.BlockSpec((_BM, D), lambda i: (i, 0)),
        out_shape=jax.ShapeDtypeStruct((N, D), jnp.float32),
    )(x, W1)


def _combine_relu_body(p_ref, b_ref, o_ref):
    o_ref[...] = jnp.maximum(p_ref[0] + p_ref[1] + b_ref[...], 0.0)


def _combine_relu(p, b1):
    return pl.pallas_call(
        _combine_relu_body,
        grid=(N // _BM,),
        in_specs=[
            pl.BlockSpec((NC, _BM, D), lambda i: (0, i, 0)),
            pl.BlockSpec((1, D), lambda i: (0, 0)),
        ],
        out_specs=pl.BlockSpec((_BM, D), lambda i: (i, 0)),
        out_shape=jax.ShapeDtypeStruct((N, D), jnp.float32),
    )(p, b1.reshape(1, D))


def _final_body(q_ref, w_ref, b_ref, o_ref):
    t = q_ref[0] + q_ref[1]
    o = jnp.dot(t, w_ref[...], preferred_element_type=jnp.float32) + b_ref[...]
    m = jnp.max(o, axis=1, keepdims=True)
    ex = jnp.exp(o - m)
    lse = jnp.log(jnp.sum(ex, axis=1, keepdims=True)) + m
    o_ref[...] = o - lse


def _final(q, W2, b2):
    return pl.pallas_call(
        _final_body,
        grid=(N // _BM,),
        in_specs=[
            pl.BlockSpec((NC, _BM, D), lambda i: (0, i, 0)),
            pl.BlockSpec((D, DC), lambda i: (0, 0)),
            pl.BlockSpec((1, DC), lambda i: (0, 0)),
        ],
        out_specs=pl.BlockSpec((_BM, DC), lambda i: (i, 0)),
        out_shape=jax.ShapeDtypeStruct((N, DC), jnp.float32),
    )(q, W2, b2.reshape(1, DC))


def kernel(x, edge_index, edge_weight, W1, b1, W2, b2):
    pad = EP - E
    pad_idx = (jnp.arange(pad, dtype=jnp.int32) * 8) % N
    src_p = jnp.concatenate([edge_index[0], pad_idx]).reshape(NW, NCH, 1, C)
    dst_p = jnp.concatenate([edge_index[1], pad_idx]).reshape(NW, NCH, 1, C)
    w_bits = lax.bitcast_convert_type(
        jnp.concatenate([edge_weight, jnp.zeros((pad,), jnp.float32)]),
        jnp.int32).reshape(NW, NCH, 1, C)
    e3 = jnp.concatenate([src_p, dst_p, w_bits], axis=2)

    support = _mm1(x, W1)
    p = _spmm_sc(support, e3)
    h = _combine_relu(p, b1)
    q = _spmm_sc(h, e3)
    return _final(q, W2, b2)


# speedup vs baseline: 3.2834x; 1.0147x over previous
import functools

import jax
import jax.numpy as jnp
from jax import lax
from jax.experimental import pallas as pl
from jax.experimental.pallas import tpu as pltpu
from jax.experimental.pallas import tpu_sc as plsc

N = 10000
E = 320000
D = 128
DC = 64

NC = 2
NS = 16
NW = NC * NS
C = 80
NB = 4
NCH = 128
EP = NW * NCH * C
RPT = N // NS

_sc_mesh = plsc.VectorSubcoreMesh(
    core_axis_name="c", subcore_axis_name="s", num_cores=NC, num_subcores=NS)


def _spmm_sc(table, e3):
    @functools.partial(
        pl.kernel,
        out_type=jax.ShapeDtypeStruct((NC, N, D), jnp.float32),
        mesh=_sc_mesh,
        scratch_types=[
            pltpu.VMEM_SHARED((N, D), jnp.float32),
        ] + [pltpu.VMEM((3, C), jnp.int32)] * NB
          + [pltpu.VMEM((C,), jnp.int32)] * NB
          + [pltpu.VMEM((C, D), jnp.float32)] * NB
          + [pltpu.SemaphoreType.DMA] * (3 * NB),
    )
    def spmm_kernel(table_hbm, e3_hbm, out_hbm, acc_sh, *bufs):
        e3_v = bufs[0:NB]
        dc_v = bufs[NB:2 * NB]
        rows_v = bufs[2 * NB:3 * NB]
        si = bufs[3 * NB:4 * NB]
        sg = bufs[4 * NB:5 * NB]
        ss = bufs[5 * NB:6 * NB]

        c = lax.axis_index("c")
        s = lax.axis_index("s")
        wid = c * NS + s

        @pl.loop(0, C)
        def _(r):
            for dd in range(D // 16):
                rows_v[0][r, pl.ds(dd * 16, 16)] = jnp.zeros((16,), jnp.float32)

        rem = RPT % C
        for k in range(RPT // C):
            pltpu.async_copy(rows_v[0], acc_sh.at[pl.ds(s * RPT + k * C, C)],
                             sg[0])
        if rem:
            pltpu.async_copy(rows_v[0].at[pl.ds(0, rem)],
                             acc_sh.at[pl.ds(s * RPT + (RPT // C) * C, rem)],
                             sg[0])
        for k in range(RPT // C):
            pltpu.make_async_copy(rows_v[0],
                                  acc_sh.at[pl.ds(s * RPT + k * C, C)],
                                  sg[0]).wait()
        if rem:
            pltpu.make_async_copy(rows_v[0].at[pl.ds(0, rem)],
                                  acc_sh.at[pl.ds(s * RPT + (RPT // C) * C,
                                                  rem)],
                                  sg[0]).wait()
        plsc.subcore_barrier()

        def process(j, b):
            pltpu.make_async_copy(
                table_hbm.at[e3_v[b].at[0]], rows_v[b], sg[b]).wait()
            for g in range(C // 16):
                sl = pl.ds(g * 16, 16)
                dc_v[b][sl] = e3_v[b][1, sl]

            @pl.loop(0, C // 16)
            def _(g):
                wv = lax.bitcast_convert_type(
                    e3_v[b][2, pl.ds(g * 16, 16)], jnp.float32)
                for k in range(16):
                    spl = jnp.full((16,), wv[k], jnp.float32)
                    e = g * 16 + k
                    for dd in range(D // 16):
                        sl2 = pl.ds(dd * 16, 16)
                        rows_v[b][e, sl2] = rows_v[b][e, sl2] * spl

            pltpu.async_copy(rows_v[b], acc_sh.at[dc_v[b]], ss[b], add=True)

            @pl.when(j + NB < NCH)
            def _():
                pltpu.async_copy(e3_hbm.at[wid, j + NB], e3_v[b], si[b])

        def refill_gather(j, b):
            @pl.when(j < NCH)
            def _():
                pltpu.make_async_copy(rows_v[b], acc_sh.at[dc_v[b]],
                                      ss[b]).wait()
                pltpu.make_async_copy(e3_hbm.at[wid, 0], e3_v[b], si[b]).wait()
                pltpu.async_copy(table_hbm.at[e3_v[b].at[0]], rows_v[b], sg[b])

        for b in range(NB):
            pltpu.async_copy(e3_hbm.at[wid, b], e3_v[b], si[b])
        for b in range(NB):
            pltpu.make_async_copy(e3_hbm.at[wid, 0], e3_v[b], si[b]).wait()
            pltpu.async_copy(table_hbm.at[e3_v[b].at[0]], rows_v[b], sg[b])

        @pl.loop(0, NCH // NB)
        def _(it):
            j0 = it * NB
            for b in range(NB):
                process(j0 + b, b)
            for b in range(NB):
                refill_gather(j0 + b + NB, b)

        for b in range(NB):
            pltpu.make_async_copy(rows_v[b], acc_sh.at[dc_v[b]], ss[b]).wait()
        plsc.subcore_barrier()

        WO = 624
        pltpu.sync_copy(acc_sh.at[pl.ds(s * WO, WO)],
                        out_hbm.at[c, pl.ds(s * WO, WO)])

        @pl.when(s == NS - 1)
        def _():
            pltpu.sync_copy(acc_sh.at[pl.ds(NS * WO, N - NS * WO)],
                            out_hbm.at[c, pl.ds(NS * WO, N - NS * WO)])

    return spmm_kernel(table, e3)


_BM = 5000


def _mm1_body(x_ref, w_ref, o_ref):
    o_ref[...] = jnp.dot(x_ref[...], w_ref[...],
                         preferred_element_type=jnp.float32)


def _mm1(x, W1):
    return pl.pallas_call(
        _mm1_body,
        grid=(N // _BM,),
        in_specs=[
            pl.BlockSpec((_BM, D), lambda i: (i, 0)),
            pl.BlockSpec((D, D), lambda i: (0, 0)),
        ],
        out_specs=pl.BlockSpec((_BM, D), lambda i: (i, 0)),
        out_shape=jax.ShapeDtypeStruct((N, D), jnp.float32),
    )(x, W1)


def _combine_relu_body(p_ref, b_ref, o_ref):
    o_ref[...] = jnp.maximum(p_ref[0] + p_ref[1] + b_ref[...], 0.0)


def _combine_relu(p, b1):
    return pl.pallas_call(
        _combine_relu_body,
        grid=(N // _BM,),
        in_specs=[
            pl.BlockSpec((NC, _BM, D), lambda i: (0, i, 0)),
            pl.BlockSpec((1, D), lambda i: (0, 0)),
        ],
        out_specs=pl.BlockSpec((_BM, D), lambda i: (i, 0)),
        out_shape=jax.ShapeDtypeStruct((N, D), jnp.float32),
    )(p, b1.reshape(1, D))


def _final_body(q_ref, w_ref, b_ref, o_ref):
    t = q_ref[0] + q_ref[1]
    o = jnp.dot(t, w_ref[...], preferred_element_type=jnp.float32) + b_ref[...]
    m = jnp.max(o, axis=1, keepdims=True)
    ex = jnp.exp(o - m)
    lse = jnp.log(jnp.sum(ex, axis=1, keepdims=True)) + m
    o_ref[...] = o - lse


def _final(q, W2, b2):
    return pl.pallas_call(
        _final_body,
        grid=(N // _BM,),
        in_specs=[
            pl.BlockSpec((NC, _BM, D), lambda i: (0, i, 0)),
            pl.BlockSpec((D, DC), lambda i: (0, 0)),
            pl.BlockSpec((1, DC), lambda i: (0, 0)),
        ],
        out_specs=pl.BlockSpec((_BM, DC), lambda i: (i, 0)),
        out_shape=jax.ShapeDtypeStruct((N, DC), jnp.float32),
    )(q, W2, b2.reshape(1, DC))


def kernel(x, edge_index, edge_weight, W1, b1, W2, b2):
    pad = EP - E
    pad_idx = (jnp.arange(pad, dtype=jnp.int32) * 8) % N
    src_p = jnp.concatenate([edge_index[0], pad_idx]).reshape(NW, NCH, 1, C)
    dst_p = jnp.concatenate([edge_index[1], pad_idx]).reshape(NW, NCH, 1, C)
    w_bits = lax.bitcast_convert_type(
        jnp.concatenate([edge_weight, jnp.zeros((pad,), jnp.float32)]),
        jnp.int32).reshape(NW, NCH, 1, C)
    e3 = jnp.concatenate([src_p, dst_p, w_bits], axis=2)

    support = _mm1(x, W1)
    p = _spmm_sc(support, e3)
    h = _combine_relu(p, b1)
    q = _spmm_sc(h, e3)
    return _final(q, W2, b2)
